# Initial kernel scaffold; baseline (speedup 1.0000x reference)
import functools

import jax
import jax.numpy as jnp
from jax import lax
from jax.experimental import pallas as pl
from jax.experimental.pallas import tpu as pltpu
from jax.experimental.pallas import tpu_sc as plsc

N = 10000
NFEAT = 128
NHID = 64
NCLASS = 40
E = 320000
GAMMA = 1.0

NPAD = 10240
PADROW = 10000
NC = 2
NS = 16
NT = NC * NS
CHUNK = 128
CH = 79
EPT = CH * CHUNK
EPAD = NT * EPT

ROWS_PER_TILE = NPAD // NS
BLK = 1280
GRID = NPAD // BLK


def _make_deg_kernel():
    mesh = plsc.VectorSubcoreMesh(core_axis_name="c", subcore_axis_name="s")

    @functools.partial(
        pl.kernel,
        out_type=jax.ShapeDtypeStruct((NT, NPAD), jnp.float32),
        mesh=mesh,
        scratch_types=[
            pltpu.VMEM((CH, CHUNK), jnp.int32),
            pltpu.VMEM((NPAD,), jnp.float32),
        ],
    )
    def deg_kernel(rowp_hbm, out_hbm, rowv, deg):
        c = lax.axis_index("c")
        s = lax.axis_index("s")
        w = c * NS + s
        pltpu.sync_copy(rowp_hbm.at[w], rowv)

        zeros16 = jnp.zeros((16,), jnp.float32)
        ones16 = jnp.ones((16,), jnp.float32)

        def zbody(i, _):
            deg[pl.ds(i * 16, 16)] = zeros16
            return 0

        lax.fori_loop(0, NPAD // 16, zbody, 0)

        def hbody(j, _):
            for k in range(CHUNK // 16):
                idx = rowv[j, pl.ds(k * 16, 16)]
                plsc.addupdate_scatter(deg, [idx], ones16)
            return 0

        lax.fori_loop(0, CH, hbody, 0)
        pltpu.sync_copy(deg, out_hbm.at[w])

    return deg_kernel


def _make_spmm_kernel(width):
    mesh = plsc.VectorSubcoreMesh(core_axis_name="c", subcore_axis_name="s")

    @functools.partial(
        pl.kernel,
        out_type=jax.ShapeDtypeStruct((NC, NPAD, width), jnp.float32),
        mesh=mesh,
        scratch_types=[
            pltpu.VMEM((CH, CHUNK), jnp.int32),
            pltpu.VMEM((CH, CHUNK), jnp.int32),
            pltpu.VMEM((CHUNK, width), jnp.float32),
            pltpu.VMEM_SHARED((NPAD, width), jnp.float32),
            pltpu.SemaphoreType.DMA,
        ],
    )
    def spmm(h_hbm, colp_hbm, rowp_hbm, zeros_hbm, out_hbm,
             colv, rowv, gbuf, accum, sem):
        c = lax.axis_index("c")
        s = lax.axis_index("s")
        w = c * NS + s
        rbase = s * ROWS_PER_TILE
        pltpu.sync_copy(zeros_hbm.at[pl.ds(rbase, ROWS_PER_TILE)],
                        accum.at[pl.ds(rbase, ROWS_PER_TILE)])
        pltpu.sync_copy(colp_hbm.at[w], colv)
        pltpu.sync_copy(rowp_hbm.at[w], rowv)
        plsc.subcore_barrier()

        def body(j, _):
            pltpu.async_copy(h_hbm.at[colv.at[j]], gbuf, sem).wait()
            pltpu.sync_copy(gbuf, accum.at[rowv.at[j]], add=True)
            return 0

        lax.fori_loop(0, CH, body, 0)
        plsc.subcore_barrier()
        pltpu.sync_copy(accum.at[pl.ds(rbase, ROWS_PER_TILE)],
                        out_hbm.at[c, pl.ds(rbase, ROWS_PER_TILE)])

    return spmm


_deg_kernel = _make_deg_kernel()
_spmm128 = _make_spmm_kernel(128)
_spmm80 = _make_spmm_kernel(80)


def _rscales(dparts_blk):
    deg = jnp.sum(dparts_blk, axis=0)
    r_half = jnp.where(deg > 0, lax.rsqrt(deg), 0.0)
    r_one = jnp.where(deg > 0, 1.0 / deg, 0.0)
    return r_half[:, None], r_one[:, None]


def _l1_body(x_ref, dparts_ref, wm_ref, ws_ref, hcat_ref):
    r_half, r_one = _rscales(dparts_ref[...])
    xb = x_ref[...]
    miu = jax.nn.elu(jnp.dot(xb, wm_ref[...],
                             preferred_element_type=jnp.float32))
    sig = jnp.maximum(jnp.dot(xb, ws_ref[...],
                              preferred_element_type=jnp.float32), 0.0)
    att = jnp.exp(-GAMMA * sig)
    h1 = miu * att * r_half
    h2 = sig * att * att * r_one
    hcat_ref[...] = jnp.concatenate([h1, h2], axis=1)


def _l1_dense(x, dparts, wm, ws):
    return pl.pallas_call(
        _l1_body,
        grid=(GRID,),
        in_specs=[
            pl.BlockSpec((BLK, NFEAT), lambda i: (i, 0)),
            pl.BlockSpec((NT, BLK), lambda i: (0, i)),
            pl.BlockSpec((NFEAT, NHID), lambda i: (0, 0)),
            pl.BlockSpec((NFEAT, NHID), lambda i: (0, 0)),
        ],
        out_specs=pl.BlockSpec((BLK, 2 * NHID), lambda i: (i, 0)),
        out_shape=jax.ShapeDtypeStruct((NPAD, 2 * NHID), jnp.float32),
    )(x, dparts, wm, ws)


def _l2_body(macc_ref, sacc_ref, dparts_ref, wm_ref, ws_ref, gcat_ref):
    r_half, r_one = _rscales(dparts_ref[...])
    miu_in = (macc_ref[0] + macc_ref[1]) * r_half
    sig_in = (sacc_ref[0] + sacc_ref[1]) * r_one
    miu2 = jax.nn.elu(jnp.dot(miu_in, wm_ref[...],
                              preferred_element_type=jnp.float32))
    sig2 = jnp.maximum(jnp.dot(sig_in, ws_ref[...],
                               preferred_element_type=jnp.float32), 0.0)
    att2 = jnp.exp(-GAMMA * sig2)
    g1 = miu2 * att2 * r_half
    g2 = sig2 * att2 * att2 * r_one
    gcat_ref[...] = jnp.concatenate([g1, g2], axis=1)


def _l2_dense(macc, sacc, dparts, wm2, ws2):
    return pl.pallas_call(
        _l2_body,
        grid=(GRID,),
        in_specs=[
            pl.BlockSpec((NC, BLK, NHID), lambda i: (0, i, 0)),
            pl.BlockSpec((NC, BLK, NHID), lambda i: (0, i, 0)),
            pl.BlockSpec((NT, BLK), lambda i: (0, i)),
            pl.BlockSpec((NHID, NCLASS), lambda i: (0, 0)),
            pl.BlockSpec((NHID, NCLASS), lambda i: (0, 0)),
        ],
        out_specs=pl.BlockSpec((BLK, 2 * NCLASS), lambda i: (i, 0)),
        out_shape=jax.ShapeDtypeStruct((NPAD, 2 * NCLASS), jnp.float32),
    )(macc, sacc, dparts, wm2, ws2)


def _out_body(pm_ref, ps_ref, dparts_ref, eps_ref, out_ref):
    r_half, r_one = _rscales(dparts_ref[...])
    mean = (pm_ref[0] + pm_ref[1]) * r_half
    sig = (ps_ref[0] + ps_ref[1]) * r_one
    o = mean + eps_ref[...] * jnp.sqrt(sig + 1e-08)
    m = jnp.max(o, axis=1, keepdims=True)
    ex = jnp.exp(o - m)
    lse = jnp.log(jnp.sum(ex, axis=1, keepdims=True))
    out_ref[...] = o - m - lse


def _out_dense(pm, ps, dparts, eps):
    return pl.pallas_call(
        _out_body,
        grid=(GRID,),
        in_specs=[
            pl.BlockSpec((NC, BLK, NCLASS), lambda i: (0, i, 0)),
            pl.BlockSpec((NC, BLK, NCLASS), lambda i: (0, i, 0)),
            pl.BlockSpec((NT, BLK), lambda i: (0, i)),
            pl.BlockSpec((BLK, NCLASS), lambda i: (i, 0)),
        ],
        out_specs=pl.BlockSpec((BLK, NCLASS), lambda i: (i, 0)),
        out_shape=jax.ShapeDtypeStruct((NPAD, NCLASS), jnp.float32),
    )(pm, ps, dparts, eps)


def kernel(x, edge_index, eps, W_miu1, W_sigma1, W_miu2, W_sigma2):
    row = edge_index[0].astype(jnp.int32)
    col = edge_index[1].astype(jnp.int32)
    padlen = EPAD - E
    pad = jnp.full((padlen,), PADROW, jnp.int32)
    rowp = jnp.concatenate([row, pad]).reshape(NT, CH, CHUNK)
    colp = jnp.concatenate([col, pad]).reshape(NT, CH, CHUNK)

    xp = jnp.zeros((NPAD, NFEAT), jnp.float32).at[:N].set(x)
    epsp = jnp.zeros((NPAD, NCLASS), jnp.float32).at[:N].set(eps)
    z128 = jnp.zeros((NPAD, 2 * NHID), jnp.float32)
    z80 = jnp.zeros((NPAD, 2 * NCLASS), jnp.float32)

    dparts = _deg_kernel(rowp)

    hcat = _l1_dense(xp, dparts, W_miu1, W_sigma1)
    acc1 = _spmm128(hcat, colp, rowp, z128)

    gcat = _l2_dense(acc1[:, :, :NHID], acc1[:, :, NHID:],
                     dparts, W_miu2, W_sigma2)
    acc2 = _spmm80(gcat, colp, rowp, z80)

    out = _out_dense(acc2[:, :, :NCLASS], acc2[:, :, NCLASS:], dparts, epsp)
    return out[:N]


# speedup vs baseline: 17.9441x; 17.9441x over previous
import functools

import jax
import jax.numpy as jnp
from jax import lax
from jax.experimental import pallas as pl
from jax.experimental.pallas import tpu as pltpu
from jax.experimental.pallas import tpu_sc as plsc

N = 10000
NFEAT = 128
NHID = 64
NCLASS = 40
E = 320000
GAMMA = 1.0

NPAD = 10240
PADROW = 10000
NC = 2
NS = 16
NT = NC * NS
CHUNK = 128
CH = 79
EPT = CH * CHUNK
EPAD = NT * EPT

ROWS_PER_TILE = NPAD // NS
BLK = 1280
GRID = NPAD // BLK


DEGW = 16


def _make_deg_kernel():
    mesh = plsc.VectorSubcoreMesh(core_axis_name="c", subcore_axis_name="s")

    @functools.partial(
        pl.kernel,
        out_type=jax.ShapeDtypeStruct((NC, NPAD, DEGW), jnp.float32),
        mesh=mesh,
        scratch_types=[
            pltpu.VMEM((CH, CHUNK), jnp.int32),
            pltpu.VMEM((CHUNK, DEGW), jnp.float32),
            pltpu.VMEM_SHARED((NPAD, DEGW), jnp.float32),
        ],
        compiler_params=pltpu.CompilerParams(use_tc_tiling_on_sc=False),
    )
    def deg_kernel(rowp_hbm, ones_hbm, zeros_hbm, out_hbm, rowv, onesb, accum):
        c = lax.axis_index("c")
        s = lax.axis_index("s")
        w = c * NS + s
        rbase = s * ROWS_PER_TILE
        pltpu.sync_copy(zeros_hbm.at[pl.ds(rbase, ROWS_PER_TILE)],
                        accum.at[pl.ds(rbase, ROWS_PER_TILE)])
        pltpu.sync_copy(rowp_hbm.at[w], rowv)
        pltpu.sync_copy(ones_hbm, onesb)
        plsc.subcore_barrier()

        def body(j, _):
            pltpu.sync_copy(onesb, accum.at[rowv.at[j]], add=True)
            return 0

        lax.fori_loop(0, CH, body, 0)
        plsc.subcore_barrier()
        pltpu.sync_copy(accum.at[pl.ds(rbase, ROWS_PER_TILE)],
                        out_hbm.at[c, pl.ds(rbase, ROWS_PER_TILE)])

    return deg_kernel


def _make_spmm_kernel(width):
    mesh = plsc.VectorSubcoreMesh(core_axis_name="c", subcore_axis_name="s")

    @functools.partial(
        pl.kernel,
        out_type=jax.ShapeDtypeStruct((NC, NPAD, width), jnp.float32),
        mesh=mesh,
        scratch_types=[
            pltpu.VMEM((CH, CHUNK), jnp.int32),
            pltpu.VMEM((CH, CHUNK), jnp.int32),
            pltpu.VMEM((CHUNK, width), jnp.float32),
            pltpu.VMEM_SHARED((NPAD, width), jnp.float32),
            pltpu.SemaphoreType.DMA,
        ],
        compiler_params=pltpu.CompilerParams(use_tc_tiling_on_sc=False),
    )
    def spmm(h_hbm, colp_hbm, rowp_hbm, zeros_hbm, out_hbm,
             colv, rowv, gbuf, accum, sem):
        c = lax.axis_index("c")
        s = lax.axis_index("s")
        w = c * NS + s
        rbase = s * ROWS_PER_TILE
        pltpu.sync_copy(zeros_hbm.at[pl.ds(rbase, ROWS_PER_TILE)],
                        accum.at[pl.ds(rbase, ROWS_PER_TILE)])
        pltpu.sync_copy(colp_hbm.at[w], colv)
        pltpu.sync_copy(rowp_hbm.at[w], rowv)
        plsc.subcore_barrier()

        def body(j, _):
            pltpu.async_copy(h_hbm.at[colv.at[j]], gbuf, sem).wait()
            pltpu.sync_copy(gbuf, accum.at[rowv.at[j]], add=True)
            return 0

        lax.fori_loop(0, CH, body, 0)
        plsc.subcore_barrier()
        pltpu.sync_copy(accum.at[pl.ds(rbase, ROWS_PER_TILE)],
                        out_hbm.at[c, pl.ds(rbase, ROWS_PER_TILE)])

    return spmm


_deg_kernel = _make_deg_kernel()
_spmm128 = _make_spmm_kernel(128)
_spmm80 = _make_spmm_kernel(80)


def _elu(v):
    return jnp.where(v > 0, v, jnp.exp(jnp.minimum(v, 0.0)) - 1.0)


def _rscales(dparts_blk):
    deg = dparts_blk[0, :, 0] + dparts_blk[1, :, 0]
    r_half = jnp.where(deg > 0, lax.rsqrt(deg), 0.0)
    r_one = jnp.where(deg > 0, 1.0 / deg, 0.0)
    return r_half[:, None], r_one[:, None]


def _l1_body(x_ref, dparts_ref, wm_ref, ws_ref, hcat_ref):
    r_half, r_one = _rscales(dparts_ref[...])
    xb = x_ref[...]
    miu = _elu(jnp.dot(xb, wm_ref[...],
                             preferred_element_type=jnp.float32))
    sig = jnp.maximum(jnp.dot(xb, ws_ref[...],
                              preferred_element_type=jnp.float32), 0.0)
    att = jnp.exp(-GAMMA * sig)
    h1 = miu * att * r_half
    h2 = sig * att * att * r_one
    hcat_ref[...] = jnp.concatenate([h1, h2], axis=1)


def _l1_dense(x, dparts, wm, ws):
    return pl.pallas_call(
        _l1_body,
        grid=(GRID,),
        in_specs=[
            pl.BlockSpec((BLK, NFEAT), lambda i: (i, 0)),
            pl.BlockSpec((NC, BLK, DEGW), lambda i: (0, i, 0)),
            pl.BlockSpec((NFEAT, NHID), lambda i: (0, 0)),
            pl.BlockSpec((NFEAT, NHID), lambda i: (0, 0)),
        ],
        out_specs=pl.BlockSpec((BLK, 2 * NHID), lambda i: (i, 0)),
        out_shape=jax.ShapeDtypeStruct((NPAD, 2 * NHID), jnp.float32),
    )(x, dparts, wm, ws)


def _l2_body(macc_ref, sacc_ref, dparts_ref, wm_ref, ws_ref, gcat_ref):
    r_half, r_one = _rscales(dparts_ref[...])
    miu_in = (macc_ref[0] + macc_ref[1]) * r_half
    sig_in = (sacc_ref[0] + sacc_ref[1]) * r_one
    miu2 = _elu(jnp.dot(miu_in, wm_ref[...],
                              preferred_element_type=jnp.float32))
    sig2 = jnp.maximum(jnp.dot(sig_in, ws_ref[...],
                               preferred_element_type=jnp.float32), 0.0)
    att2 = jnp.exp(-GAMMA * sig2)
    g1 = miu2 * att2 * r_half
    g2 = sig2 * att2 * att2 * r_one
    gcat_ref[...] = jnp.concatenate([g1, g2], axis=1)


def _l2_dense(macc, sacc, dparts, wm2, ws2):
    return pl.pallas_call(
        _l2_body,
        grid=(GRID,),
        in_specs=[
            pl.BlockSpec((NC, BLK, NHID), lambda i: (0, i, 0)),
            pl.BlockSpec((NC, BLK, NHID), lambda i: (0, i, 0)),
            pl.BlockSpec((NC, BLK, DEGW), lambda i: (0, i, 0)),
            pl.BlockSpec((NHID, NCLASS), lambda i: (0, 0)),
            pl.BlockSpec((NHID, NCLASS), lambda i: (0, 0)),
        ],
        out_specs=pl.BlockSpec((BLK, 2 * NCLASS), lambda i: (i, 0)),
        out_shape=jax.ShapeDtypeStruct((NPAD, 2 * NCLASS), jnp.float32),
    )(macc, sacc, dparts, wm2, ws2)


def _out_body(pm_ref, ps_ref, dparts_ref, eps_ref, out_ref):
    r_half, r_one = _rscales(dparts_ref[...])
    mean = (pm_ref[0] + pm_ref[1]) * r_half
    sig = (ps_ref[0] + ps_ref[1]) * r_one
    o = mean + eps_ref[...] * jnp.sqrt(sig + 1e-08)
    m = jnp.max(o, axis=1, keepdims=True)
    ex = jnp.exp(o - m)
    lse = jnp.log(jnp.sum(ex, axis=1, keepdims=True))
    out_ref[...] = o - m - lse


def _out_dense(pm, ps, dparts, eps):
    return pl.pallas_call(
        _out_body,
        grid=(GRID,),
        in_specs=[
            pl.BlockSpec((NC, BLK, NCLASS), lambda i: (0, i, 0)),
            pl.BlockSpec((NC, BLK, NCLASS), lambda i: (0, i, 0)),
            pl.BlockSpec((NC, BLK, DEGW), lambda i: (0, i, 0)),
            pl.BlockSpec((BLK, NCLASS), lambda i: (i, 0)),
        ],
        out_specs=pl.BlockSpec((BLK, NCLASS), lambda i: (i, 0)),
        out_shape=jax.ShapeDtypeStruct((NPAD, NCLASS), jnp.float32),
    )(pm, ps, dparts, eps)


def kernel(x, edge_index, eps, W_miu1, W_sigma1, W_miu2, W_sigma2):
    row = edge_index[0].astype(jnp.int32)
    col = edge_index[1].astype(jnp.int32)
    padlen = EPAD - E
    pad = jnp.full((padlen,), PADROW, jnp.int32)
    rowp = jnp.concatenate([row, pad]).reshape(NT, CH, CHUNK)
    colp = jnp.concatenate([col, pad]).reshape(NT, CH, CHUNK)

    xp = jnp.zeros((NPAD, NFEAT), jnp.float32).at[:N].set(x)
    epsp = jnp.zeros((NPAD, NCLASS), jnp.float32).at[:N].set(eps)
    z128 = jnp.zeros((NPAD, 2 * NHID), jnp.float32)
    z80 = jnp.zeros((NPAD, 2 * NCLASS), jnp.float32)

    ones_deg = jnp.ones((CHUNK, DEGW), jnp.float32)
    zeros_deg = jnp.zeros((NPAD, DEGW), jnp.float32)
    dparts = _deg_kernel(rowp, ones_deg, zeros_deg)

    hcat = _l1_dense(xp, dparts, W_miu1, W_sigma1)
    acc1 = _spmm128(hcat, colp, rowp, z128)

    gcat = _l2_dense(acc1[:, :, :NHID], acc1[:, :, NHID:],
                     dparts, W_miu2, W_sigma2)
    acc2 = _spmm80(gcat, colp, rowp, z80)

    out = _out_dense(acc2[:, :, :NCLASS], acc2[:, :, NCLASS:], dparts, epsp)
    return out[:N]
